# baseline (speedup 1.0000x reference)
import functools

import jax
import jax.numpy as jnp
from jax import lax
from jax.experimental import pallas as pl
from jax.experimental.pallas import tpu as pltpu
from jax.experimental.pallas import tpu_sc as plsc

_WIN = 256
_NBUF = 2


def _make_pool(B, L, D):
    D2 = 2 * D
    info = plsc.get_sparse_core_info()
    NC, NS, LN = info.num_cores, info.num_subcores, info.num_lanes
    NW = NC * NS
    bpw = B // NW
    rpw = bpw * L
    CH, NBUF = _WIN, _NBUF
    nstr = CH // LN
    nch = rpw // CH
    ngrp = nch // NBUF
    ACC_H = NS * bpw
    mesh = plsc.VectorSubcoreMesh(core_axis_name="c", subcore_axis_name="s")

    @functools.partial(
        pl.kernel,
        mesh=mesh,
        out_type=jax.ShapeDtypeStruct((B, D), jnp.float32),
        scratch_types=[
            pltpu.VMEM((nch, 1, CH), jnp.int32),
            pltpu.VMEM((nch, 1, CH), jnp.int32),
            pltpu.VMEM((NBUF, CH, D2), jnp.float32),
            pltpu.VMEM((bpw, D), jnp.float32),
            pltpu.VMEM_SHARED((2 * ACC_H, D2), jnp.float32),
            pltpu.SemaphoreType.DMA((NBUF,)),
            pltpu.SemaphoreType.DMA((NBUF,)),
            pltpu.SemaphoreType.DMA,
        ],
    )
    def pool(x_hbm, emb_hbm, out_hbm,
             idx_v, dst_v, rows_v, comb_v, acc, gsem, ssem, csem):
        cid = lax.axis_index("c")
        sid = lax.axis_index("s")
        wid = sid * NC + cid

        cp0 = pltpu.async_copy(x_hbm.at[wid], idx_v, csem)

        zero2 = jnp.zeros((LN,), jnp.float32)

        def zbody(r, carry):
            for k in range(D2 // LN):
                rows_v[0, r, pl.ds(LN * k, LN)] = zero2
            return carry

        lax.fori_loop(0, bpw, zbody, 0)
        pltpu.sync_copy(rows_v.at[0, pl.ds(0, bpw)],
                        acc.at[pl.ds(sid * bpw, bpw)])
        pltpu.sync_copy(rows_v.at[0, pl.ds(0, bpw)],
                        acc.at[pl.ds(ACC_H + sid * bpw, bpw)])
        cp0.wait()

        base = sid * bpw

        def tbody(c, carry):
            for j in range(nstr):
                v = idx_v[c, 0, pl.ds(LN * j, LN)]
                loc = lax.iota(jnp.int32, LN) + ((LN * j) % bpw + base)
                dst_v[c, 0, pl.ds(LN * j, LN)] = (v & 1) * ACC_H + loc
                idx_v[c, 0, pl.ds(LN * j, LN)] = lax.shift_right_logical(v, 1)
            return carry

        lax.fori_loop(0, nch, tbody, 0)

        def issue_window(c, b):
            for j in range(nstr):
                iv = idx_v[c, 0, pl.ds(LN * j, LN)]
                pltpu.async_copy(
                    emb_hbm.at[iv], rows_v.at[b, pl.ds(LN * j, LN)],
                    gsem.at[b])

        def drain_window(c, b):
            pltpu.make_async_copy(
                emb_hbm.at[idx_v.at[c, 0]], rows_v.at[b], gsem.at[b]).wait()

        for b in range(NBUF):
            issue_window(b, b)

        def grp(g, carry):
            c0 = g * NBUF
            cps = []
            for b in range(NBUF):
                drain_window(c0 + b, b)
                cps.append(pltpu.async_copy(
                    rows_v.at[b], acc.at[dst_v.at[c0 + b, 0]], ssem.at[b],
                    add=True))
            for b in range(NBUF):
                cps[b].wait()

                @pl.when(g < ngrp - 1)
                def _():
                    issue_window(c0 + NBUF + b, b)
            return carry

        lax.fori_loop(0, ngrp, grp, 0)

        pltpu.sync_copy(acc.at[pl.ds(sid * bpw, bpw)],
                        rows_v.at[0, pl.ds(0, bpw)])
        pltpu.sync_copy(acc.at[pl.ds(ACC_H + sid * bpw, bpw)],
                        rows_v.at[1, pl.ds(0, bpw)])

        def cbody(r, carry):
            for k in range(D // LN):
                lo = rows_v[0, r, pl.ds(LN * k, LN)]
                hi = rows_v[1, r, pl.ds(D + LN * k, LN)]
                comb_v[r, pl.ds(LN * k, LN)] = lo + hi
            return carry

        lax.fori_loop(0, bpw, cbody, 0)
        pltpu.sync_copy(comb_v, out_hbm.at[pl.ds(wid * bpw, bpw)])

    return pool


def _head(pooled, W, b2, scale):
    B, D = pooled.shape
    OUT = W.shape[1]

    def body(p_ref, w_ref, b_ref, o_ref):
        h = jnp.maximum(p_ref[...] * scale, 0.0)
        o_ref[...] = lax.dot_general(
            h, w_ref[...], (((1,), (0,)), ((), ())),
            preferred_element_type=jnp.float32) + b_ref[...]

    return pl.pallas_call(
        body,
        out_shape=jax.ShapeDtypeStruct((B, OUT), jnp.float32),
    )(pooled, W, b2)


def kernel(x, emb, W, b):
    B, L = x.shape
    V, D = emb.shape
    info = plsc.get_sparse_core_info()
    NC, NS = info.num_cores, info.num_subcores
    NW = NC * NS
    Bh = B // 2
    bpw = Bh // NW
    nch = bpw * L // _WIN

    emb2 = emb.reshape(V // 2, 2 * D)

    def prep(xh):
        return (xh.astype(jnp.int32).reshape(NW, bpw, L)
                .transpose(0, 2, 1).reshape(NW, nch, 1, _WIN))

    pool = _make_pool(Bh, L, D)
    pooled = jnp.concatenate(
        [pool(prep(x[:Bh]), emb2), pool(prep(x[Bh:]), emb2)], axis=0)
    return _head(pooled, W, b.reshape(1, -1), 1.0 / L)
